# baseline (speedup 1.0000x reference)
import jax
import jax.numpy as jnp
from jax.experimental import pallas as pl
from jax.experimental.pallas import tpu as pltpu

IMG_ROWS, IMG_COLS, NUM_CHAN = 8, 9, 4
C1, C2, C3, C4 = 64, 128, 256, 64
FC_OUT = 512
PH, PW = IMG_ROWS // 2, IMG_COLS // 2
FC_IN = C4 * PH * PW

B_TILE = 32
HF, WF = IMG_ROWS + 4, IMG_COLS + 4
MF = HF * WF
X = MF * B_TILE
KW1 = 5 * NUM_CHAN
G0 = 2 * WF * B_TILE
EXT0 = X + 2 * G0
HF2, WF2 = IMG_ROWS + 2, IMG_COLS + 2
MF2 = HF2 * WF2
X2 = MF2 * B_TILE
GA = (WF2 + 1) * B_TILE
EXTA = X2 + 2 * GA


def _fused_cnn_kernel(x_ref, mask1_ref, mask2_ref, w1_ref, b1_ref, w2_ref,
                      b2_ref, w3_ref, b3_ref, w4_ref, b4_ref, wfc_ref,
                      bfc_ref, out_ref, xdw_ref, xin_ref, xa_ref, xb_ref,
                      feat_ref):
    B = B_TILE

    xdw_ref[0:2 * B, :] = jnp.zeros((2 * B, KW1), jnp.bfloat16)
    xdw_ref[X - 2 * B:X, :] = jnp.zeros((2 * B, KW1), jnp.bfloat16)
    for dw in range(5):
        s = (dw - 2) * B
        lo, hi = max(0, -s), X + min(0, -s)
        xdw_ref[lo:hi, dw * NUM_CHAN:(dw + 1) * NUM_CHAN] = \
            x_ref[lo + s:hi + s, 0:NUM_CHAN]

    xin_ref[G0:G0 + G0, :] = jnp.zeros((G0, 5 * KW1), jnp.bfloat16)
    xin_ref[G0 + X - G0:G0 + X, :] = jnp.zeros((G0, 5 * KW1), jnp.bfloat16)
    xblk = xdw_ref[...]
    for k in range(5):
        off = (k - 2) * WF * B
        xin_ref[G0 - off:G0 - off + X, k * KW1:(k + 1) * KW1] = xblk
    a1 = jnp.dot(xin_ref[G0:G0 + X, :], w1_ref[...],
                 preferred_element_type=jnp.float32)
    a1 = jnp.maximum(a1 + b1_ref[...], 0.0).astype(jnp.bfloat16)
    a1 = a1 * mask1_ref[...]

    a1c = jnp.concatenate(
        [a1[((rr2 + 1) * WF + 1) * B:((rr2 + 1) * WF + 1) * B + WF2 * B, :]
         for rr2 in range(HF2)], axis=0)

    def pack9(dst_ref, a, cin):
        dst_ref[GA:GA + GA, :] = jnp.zeros((GA, 9 * cin), jnp.bfloat16)
        dst_ref[GA + X2 - GA:GA + X2, :] = jnp.zeros((GA, 9 * cin),
                                                     jnp.bfloat16)
        for dh in range(3):
            for dw in range(3):
                j = dh * 3 + dw
                off = ((dh - 1) * WF2 + (dw - 1)) * B
                dst_ref[GA - off:GA - off + X2, j * cin:(j + 1) * cin] = a

    pack9(xa_ref, a1c, C1)
    a2 = jnp.dot(xa_ref[GA:GA + X2, :], w2_ref[...],
                 preferred_element_type=jnp.float32)
    a2 = jnp.maximum(a2 + b2_ref[...], 0.0).astype(jnp.bfloat16)
    a2 = a2 * mask2_ref[...]

    pack9(xb_ref, a2, C2)
    a3 = jnp.dot(xb_ref[GA:GA + X2, :], w3_ref[...],
                 preferred_element_type=jnp.float32)
    a3 = jnp.maximum(a3 + b3_ref[...], 0.0).astype(jnp.bfloat16)

    a4 = jnp.maximum(jnp.dot(a3, w4_ref[...], preferred_element_type=jnp.float32)
                     + b4_ref[...], 0.0)
    xa_ref[GA:GA + X2, 0:C4] = a4.astype(jnp.bfloat16)

    for ph in range(PH):
        for pw in range(PW):
            p00 = (1 + 2 * ph) * WF2 + (1 + 2 * pw)
            r = GA + p00 * B
            v = jnp.maximum(
                jnp.maximum(xa_ref[r:r + B, 0:C4],
                            xa_ref[r + B:r + 2 * B, 0:C4]),
                jnp.maximum(xa_ref[r + WF2 * B:r + WF2 * B + B, 0:C4],
                            xa_ref[r + (WF2 + 1) * B:r + (WF2 + 1) * B + B,
                                   0:C4]))
            s = ph * PW + pw
            feat_ref[:, s * C4:(s + 1) * C4] = v
    out = jnp.dot(feat_ref[...], wfc_ref[...], preferred_element_type=jnp.float32)
    out_ref[...] = jnp.maximum(out + bfc_ref[...], 0.0)


def kernel(w1, b1, w2, b2, w3, b3, w4, b4, fc_w, fc_b, x):
    N = x.shape[0]
    G = -(-N // B_TILE)
    N_pad = G * B_TILE
    x = jnp.pad(x.astype(jnp.bfloat16),
                ((0, N_pad - N), (0, 0), (0, 0), (0, 0)))

    sel = jnp.eye(B_TILE * NUM_CHAN,
                  dtype=jnp.bfloat16).reshape(B_TILE, NUM_CHAN,
                                              B_TILE * NUM_CHAN)
    x_q = jnp.einsum('gbcp,bcj->gpj',
                     x.reshape(G, B_TILE, NUM_CHAN, IMG_ROWS * IMG_COLS),
                     sel,
                     preferred_element_type=jnp.float32).astype(jnp.bfloat16)
    x_fr = jnp.pad(x_q.reshape(G, IMG_ROWS, IMG_COLS, B_TILE, NUM_CHAN),
                   ((0, 0), (2, 2), (2, 2), (0, 0),
                    (0, 8 - NUM_CHAN)))
    x_raw = x_fr.reshape(G, X, 8)

    def interior_mask(hf, wf, ring, lanes):
        mf = hf * wf
        rr = jnp.arange(mf, dtype=jnp.int32) // wf
        cc = jnp.arange(mf, dtype=jnp.int32) % wf
        m = ((rr >= ring) & (rr < ring + IMG_ROWS) &
             (cc >= ring) & (cc < ring + IMG_COLS)).astype(jnp.bfloat16)
        return jnp.broadcast_to(m[:, None, None],
                                (mf, B_TILE, lanes)).reshape(mf * B_TILE, lanes)

    mask1 = interior_mask(HF, WF, 2, C1)
    mask2 = interior_mask(HF2, WF2, 1, C2)

    bf = jnp.bfloat16
    w1m = (jnp.transpose(w1, (2, 3, 1, 0))
              .reshape(5 * KW1, C1).astype(bf))

    def packw(w, cin, cout):
        return jnp.transpose(w, (2, 3, 1, 0)).reshape(9 * cin, cout).astype(bf)

    w2m = packw(w2, C1, C2)
    w3m = packw(w3, C2, C3)
    w4m = w4[:, :, 0, 0].T.astype(bf)
    wfc = jnp.transpose(fc_w.reshape(FC_OUT, C4, PH, PW),
                        (2, 3, 1, 0)).reshape(FC_IN, FC_OUT).astype(bf)
    b1r, b2r, b3r, b4r = (v[None, :] for v in (b1, b2, b3, b4))
    bfc = fc_b[None, :]

    def full(a):
        return pl.BlockSpec(a.shape, lambda g, _nd=a.ndim: (0,) * _nd)

    flops = (G * 2 * (X * 5 * KW1 * C1
                      + X2 * (9 * C1 * C2 + 9 * C2 * C3 + C3 * C4))
             + G * 2 * B_TILE * FC_IN * FC_OUT)
    bytes_accessed = 2 * (x_raw.size + mask1.size + mask2.size + w1m.size
                          + w2m.size + w3m.size + w4m.size + wfc.size) \
        + 4 * N_pad * FC_OUT

    out = pl.pallas_call(
        _fused_cnn_kernel,
        out_shape=jax.ShapeDtypeStruct((N_pad, FC_OUT), jnp.float32),
        grid=(G,),
        in_specs=[
            pl.BlockSpec((None, X, 8), lambda g: (g, 0, 0)),
            full(mask1), full(mask2),
            full(w1m), full(b1r),
            full(w2m), full(b2r),
            full(w3m), full(b3r),
            full(w4m), full(b4r),
            full(wfc), full(bfc),
        ],
        out_specs=pl.BlockSpec((B_TILE, FC_OUT), lambda g: (g, 0)),
        scratch_shapes=[
            pltpu.VMEM((X, KW1), jnp.bfloat16),
            pltpu.VMEM((EXT0, 5 * KW1), jnp.bfloat16),
            pltpu.VMEM((EXTA, 9 * C1), jnp.bfloat16),
            pltpu.VMEM((EXTA, 9 * C2), jnp.bfloat16),
            pltpu.VMEM((B_TILE, FC_IN), jnp.bfloat16),
        ],
        compiler_params=pltpu.CompilerParams(
            dimension_semantics=("parallel",),
            vmem_limit_bytes=48 * 1024 * 1024),
        cost_estimate=pl.CostEstimate(flops=flops, transcendentals=0,
                                      bytes_accessed=bytes_accessed),
    )(x_raw, mask1, mask2, w1m, b1r, w2m, b2r, w3m, b3r, w4m, b4r, wfc, bfc)
    return out[:N]


# speedup vs baseline: 1.3762x; 1.3762x over previous
import jax
import jax.numpy as jnp
from jax.experimental import pallas as pl
from jax.experimental.pallas import tpu as pltpu

IMG_ROWS, IMG_COLS, NUM_CHAN = 8, 9, 4
C1, C2, C3, C4 = 64, 128, 256, 64
FC_OUT = 512
PH, PW = IMG_ROWS // 2, IMG_COLS // 2
FC_IN = C4 * PH * PW

B_TILE = 32
HF, WF = IMG_ROWS + 4, IMG_COLS + 4
MF = HF * WF
X = MF * B_TILE
KW1 = 5 * NUM_CHAN
G0 = 2 * WF * B_TILE
EXT0 = X + 2 * G0
HF2, WF2 = IMG_ROWS + 2, IMG_COLS + 2
MF2 = HF2 * WF2
X2 = MF2 * B_TILE
GA = (WF2 + 1) * B_TILE
EXTA = X2 + 2 * GA


def _fused_cnn_kernel(x_ref, mask1_ref, mask2_ref, w1_ref, b1_ref, w2_ref,
                      b2_ref, w3_ref, b3_ref, w4_ref, b4_ref, wfc_ref,
                      bfc_ref, out_ref, xin_ref, xa_ref, xb_ref, feat_ref):
    B = B_TILE

    xin_ref[G0:G0 + G0, :] = jnp.zeros((G0, 5 * KW1), jnp.bfloat16)
    xin_ref[G0 + X - G0:G0 + X, :] = jnp.zeros((G0, 5 * KW1), jnp.bfloat16)
    xblk = x_ref[...]
    for k in range(5):
        off = (k - 2) * WF * B
        xin_ref[G0 - off:G0 - off + X, k * KW1:(k + 1) * KW1] = xblk
    a1 = jnp.dot(xin_ref[G0:G0 + X, :], w1_ref[...],
                 preferred_element_type=jnp.float32)
    a1 = jnp.maximum(a1 + b1_ref[...], 0.0).astype(jnp.bfloat16)
    a1 = a1 * mask1_ref[...]

    a1c = jnp.concatenate(
        [a1[((rr2 + 1) * WF + 1) * B:((rr2 + 1) * WF + 1) * B + WF2 * B, :]
         for rr2 in range(HF2)], axis=0)

    def pack9(dst_ref, a, cin):
        dst_ref[GA:GA + GA, :] = jnp.zeros((GA, 9 * cin), jnp.bfloat16)
        dst_ref[GA + X2 - GA:GA + X2, :] = jnp.zeros((GA, 9 * cin),
                                                     jnp.bfloat16)
        for dh in range(3):
            for dw in range(3):
                j = dh * 3 + dw
                off = ((dh - 1) * WF2 + (dw - 1)) * B
                dst_ref[GA - off:GA - off + X2, j * cin:(j + 1) * cin] = a

    pack9(xa_ref, a1c, C1)
    a2 = jnp.dot(xa_ref[GA:GA + X2, :], w2_ref[...],
                 preferred_element_type=jnp.float32)
    a2 = jnp.maximum(a2 + b2_ref[...], 0.0).astype(jnp.bfloat16)
    a2 = a2 * mask2_ref[...]

    pack9(xb_ref, a2, C2)
    a3 = jnp.dot(xb_ref[GA:GA + X2, :], w3_ref[...],
                 preferred_element_type=jnp.float32)
    a3 = jnp.maximum(a3 + b3_ref[...], 0.0).astype(jnp.bfloat16)

    a4 = jnp.maximum(jnp.dot(a3, w4_ref[...], preferred_element_type=jnp.float32)
                     + b4_ref[...], 0.0)
    xa_ref[GA:GA + X2, 0:C4] = a4.astype(jnp.bfloat16)

    for ph in range(PH):
        for pw in range(PW):
            p00 = (1 + 2 * ph) * WF2 + (1 + 2 * pw)
            r = GA + p00 * B
            v = jnp.maximum(
                jnp.maximum(xa_ref[r:r + B, 0:C4],
                            xa_ref[r + B:r + 2 * B, 0:C4]),
                jnp.maximum(xa_ref[r + WF2 * B:r + WF2 * B + B, 0:C4],
                            xa_ref[r + (WF2 + 1) * B:r + (WF2 + 1) * B + B,
                                   0:C4]))
            s = ph * PW + pw
            feat_ref[:, s * C4:(s + 1) * C4] = v
    out = jnp.dot(feat_ref[...], wfc_ref[...], preferred_element_type=jnp.float32)
    out_ref[...] = jnp.maximum(out + bfc_ref[...], 0.0)


def kernel(w1, b1, w2, b2, w3, b3, w4, b4, fc_w, fc_b, x):
    N = x.shape[0]
    G = -(-N // B_TILE)
    N_pad = G * B_TILE
    x = jnp.pad(x.astype(jnp.bfloat16),
                ((0, N_pad - N), (0, 0), (0, 0), (0, 0)))

    sel = jnp.eye(B_TILE * NUM_CHAN,
                  dtype=jnp.bfloat16).reshape(B_TILE, NUM_CHAN,
                                              B_TILE * NUM_CHAN)
    x_q = jnp.einsum('gbcp,bcj->gpj',
                     x.reshape(G, B_TILE, NUM_CHAN, IMG_ROWS * IMG_COLS),
                     sel,
                     preferred_element_type=jnp.float32).astype(jnp.bfloat16)
    x_gw = jnp.pad(x_q.reshape(G, IMG_ROWS, IMG_COLS, B_TILE, NUM_CHAN),
                   ((0, 0), (2, 2), (4, 4), (0, 0), (0, 0)))
    dw_taps = [x_gw[:, :, dw:dw + WF] for dw in range(5)]
    x_dw = jnp.concatenate(dw_taps, axis=-1)
    x_raw = x_dw.reshape(G, X, KW1)

    def interior_mask(hf, wf, ring, lanes):
        mf = hf * wf
        rr = jnp.arange(mf, dtype=jnp.int32) // wf
        cc = jnp.arange(mf, dtype=jnp.int32) % wf
        m = ((rr >= ring) & (rr < ring + IMG_ROWS) &
             (cc >= ring) & (cc < ring + IMG_COLS)).astype(jnp.bfloat16)
        return jnp.broadcast_to(m[:, None, None],
                                (mf, B_TILE, lanes)).reshape(mf * B_TILE, lanes)

    mask1 = interior_mask(HF, WF, 2, C1)
    mask2 = interior_mask(HF2, WF2, 1, C2)

    bf = jnp.bfloat16
    w1m = (jnp.transpose(w1, (2, 3, 1, 0))
              .reshape(5 * KW1, C1).astype(bf))

    def packw(w, cin, cout):
        return jnp.transpose(w, (2, 3, 1, 0)).reshape(9 * cin, cout).astype(bf)

    w2m = packw(w2, C1, C2)
    w3m = packw(w3, C2, C3)
    w4m = w4[:, :, 0, 0].T.astype(bf)
    wfc = jnp.transpose(fc_w.reshape(FC_OUT, C4, PH, PW),
                        (2, 3, 1, 0)).reshape(FC_IN, FC_OUT).astype(bf)
    b1r, b2r, b3r, b4r = (v[None, :] for v in (b1, b2, b3, b4))
    bfc = fc_b[None, :]

    def full(a):
        return pl.BlockSpec(a.shape, lambda g, _nd=a.ndim: (0,) * _nd)

    flops = (G * 2 * (X * 5 * KW1 * C1
                      + X2 * (9 * C1 * C2 + 9 * C2 * C3 + C3 * C4))
             + G * 2 * B_TILE * FC_IN * FC_OUT)
    bytes_accessed = 2 * (x_raw.size + mask1.size + mask2.size + w1m.size
                          + w2m.size + w3m.size + w4m.size + wfc.size) \
        + 4 * N_pad * FC_OUT

    out = pl.pallas_call(
        _fused_cnn_kernel,
        out_shape=jax.ShapeDtypeStruct((N_pad, FC_OUT), jnp.float32),
        grid=(G,),
        in_specs=[
            pl.BlockSpec((None, X, KW1), lambda g: (g, 0, 0)),
            full(mask1), full(mask2),
            full(w1m), full(b1r),
            full(w2m), full(b2r),
            full(w3m), full(b3r),
            full(w4m), full(b4r),
            full(wfc), full(bfc),
        ],
        out_specs=pl.BlockSpec((B_TILE, FC_OUT), lambda g: (g, 0)),
        scratch_shapes=[
            pltpu.VMEM((EXT0, 5 * KW1), jnp.bfloat16),
            pltpu.VMEM((EXTA, 9 * C1), jnp.bfloat16),
            pltpu.VMEM((EXTA, 9 * C2), jnp.bfloat16),
            pltpu.VMEM((B_TILE, FC_IN), jnp.bfloat16),
        ],
        compiler_params=pltpu.CompilerParams(
            dimension_semantics=("parallel",),
            vmem_limit_bytes=48 * 1024 * 1024),
        cost_estimate=pl.CostEstimate(flops=flops, transcendentals=0,
                                      bytes_accessed=bytes_accessed),
    )(x_raw, mask1, mask2, w1m, b1r, w2m, b2r, w3m, b3r, w4m, b4r, wfc, bfc)
    return out[:N]
